# baseline (speedup 1.0000x reference)
import jax
import jax.numpy as jnp
import numpy as np
from jax.experimental import pallas as pl
from jax.experimental.pallas import tpu as pltpu

INPUT_SIZE = 16384
OUTPUT_SIZE = 1024
N_UNITS = 64
CHANNEL_SIZE = 64
ENV_DIMS = 32
REACH = 1.0
GARBAGE_DECAY = 0.9
REACH_THRESHOLD = float(np.clip(1.0 - REACH, -1.0, 1.0))
GARBAGE_SCALE = float(np.clip(1.0 - GARBAGE_DECAY, 0.0, 1.0))
CAPACITY = INPUT_SIZE + N_UNITS
N_QUERIES = N_UNITS + OUTPUT_SIZE

NTILES = 4
TILE = CAPACITY // NTILES
HALF = TILE // 2
C = CHANNEL_SIZE
E = ENV_DIMS


def _normalize(x):
    s = jnp.sum(x * x, axis=1, keepdims=True)
    return x * jax.lax.rsqrt(jnp.clip(s, 1e-24, None))


def _fused_kernel(state_ref, tails_ref, eh_ref, uh_ref, w_ref, b_ref,
                  db_ref, exit_ref, gsum_ref,
                  q_ref, dist_acc, gsum_acc):
    i = pl.program_id(0)

    @pl.when(i == 0)
    def _init_queries():
        q_ref[...] = jnp.concatenate(
            [_normalize(uh_ref[...]), _normalize(eh_ref[...])], axis=0)

    base = i * TILE
    rows2 = 2 * jax.lax.broadcasted_iota(jnp.int32, (HALF, 1), 0) + base
    valid = rows2 < INPUT_SIZE

    tp = jnp.where(valid, tails_ref[...], 0.0)
    zn = jnp.concatenate(
        [_normalize(tp[:, :E]), _normalize(tp[:, E:])], axis=1)

    sp = jnp.where(valid, state_ref[...], 0.0)

    q = q_ref[...]
    zq = jnp.zeros_like(q)
    sims_e = jax.lax.dot_general(
        jnp.concatenate([q, zq], axis=1), zn, (((1,), (1,)), ((), ())),
        preferred_element_type=jnp.float32)
    sims_o = jax.lax.dot_general(
        jnp.concatenate([zq, q], axis=1), zn, (((1,), (1,)), ((), ())),
        preferred_element_type=jnp.float32)
    infl_e = jnp.maximum(sims_e - REACH_THRESHOLD, 0.0)
    infl_o = jnp.maximum(sims_o - REACH_THRESHOLD, 0.0)

    dp_e = jnp.dot(infl_e, sp, preferred_element_type=jnp.float32)
    dp_o = jnp.dot(infl_o, sp, preferred_element_type=jnp.float32)
    dist_part = dp_e[:, :C] + dp_o[:, C:]

    st_e = sp[:, :C]
    st_o = sp[:, C:]
    t_e = st_e * (jnp.sum(infl_e, axis=0) - 1.0)[:, None]
    t_o = st_o * (jnp.sum(infl_o, axis=0) - 1.0)[:, None]
    db_ref[...] = jnp.concatenate(
        [st_e - GARBAGE_SCALE * t_e, st_o - GARBAGE_SCALE * t_o], axis=1)
    g_part = -(jnp.sum(t_e, axis=0, keepdims=True)
               + jnp.sum(t_o, axis=0, keepdims=True))

    @pl.when(i == 0)
    def _init():
        dist_acc[...] = dist_part
        gsum_acc[...] = g_part

    @pl.when(i > 0)
    def _accum():
        dist_acc[...] += dist_part
        gsum_acc[...] += g_part

    @pl.when(i == NTILES - 1)
    def _finish():
        dist = dist_acc[...]
        exit_ref[...] = dist[N_UNITS:]
        gsum_ref[...] = gsum_acc[...]
        unit_in = dist[:N_UNITS]
        w = w_ref[...]
        prod = jnp.sum(unit_in[:, :, None] * w, axis=1)
        unit_out = unit_in + jnp.maximum(prod + b_ref[...], 0.0)
        r = jax.lax.broadcasted_iota(jnp.int32, (N_UNITS // 2, N_UNITS), 0)
        c = jax.lax.broadcasted_iota(jnp.int32, (N_UNITS // 2, N_UNITS), 1)
        sel_e = (c == 2 * r).astype(jnp.float32)
        sel_o = (c == 2 * r + 1).astype(jnp.float32)
        u_e = jnp.dot(sel_e, unit_out, preferred_element_type=jnp.float32)
        u_o = jnp.dot(sel_o, unit_out, preferred_element_type=jnp.float32)
        db_ref[HALF - N_UNITS // 2:, :] = jnp.concatenate([u_e, u_o], axis=1)


@jax.jit
def _run(state, input_tails, exit_heads, unit_heads, unit_W, unit_b):
    state_p = state.reshape(INPUT_SIZE // 2, 2 * C)
    tails_p = input_tails.reshape(INPUT_SIZE // 2, 2 * E)

    out_shapes = (
        jax.ShapeDtypeStruct((CAPACITY // 2, 2 * C), jnp.float32),
        jax.ShapeDtypeStruct((OUTPUT_SIZE, C), jnp.float32),
        jax.ShapeDtypeStruct((1, C), jnp.float32),
    )
    db_p, exit_out, gsum = pl.pallas_call(
        _fused_kernel,
        grid=(NTILES,),
        in_specs=[
            pl.BlockSpec((HALF, 2 * C), lambda i: (i, 0)),
            pl.BlockSpec((HALF, 2 * E), lambda i: (i, 0)),
            pl.BlockSpec((OUTPUT_SIZE, E), lambda i: (0, 0)),
            pl.BlockSpec((N_UNITS, E), lambda i: (0, 0)),
            pl.BlockSpec((N_UNITS, C, C), lambda i: (0, 0, 0)),
            pl.BlockSpec((N_UNITS, C), lambda i: (0, 0)),
        ],
        out_specs=[
            pl.BlockSpec((HALF, 2 * C), lambda i: (i, 0)),
            pl.BlockSpec((OUTPUT_SIZE, C), lambda i: (0, 0)),
            pl.BlockSpec((1, C), lambda i: (0, 0)),
        ],
        out_shape=out_shapes,
        scratch_shapes=[
            pltpu.VMEM((N_QUERIES, E), jnp.float32),
            pltpu.VMEM((N_QUERIES, C), jnp.float32),
            pltpu.VMEM((1, C), jnp.float32),
        ],
    )(state_p, tails_p, exit_heads, unit_heads, unit_W, unit_b)
    db_data = db_p.reshape(CAPACITY, C)
    return exit_out, gsum.reshape(C), db_data


def kernel(state, input_tails, exit_heads, unit_heads, unit_tails, unit_W,
           unit_b, step=1):
    del unit_tails, step
    return _run(state, input_tails, exit_heads, unit_heads, unit_W, unit_b)


# speedup vs baseline: 1.3800x; 1.3800x over previous
import jax
import jax.numpy as jnp
import numpy as np
from jax.experimental import pallas as pl
from jax.experimental.pallas import tpu as pltpu

INPUT_SIZE = 16384
OUTPUT_SIZE = 1024
N_UNITS = 64
CHANNEL_SIZE = 64
ENV_DIMS = 32
REACH = 1.0
GARBAGE_DECAY = 0.9
REACH_THRESHOLD = float(np.clip(1.0 - REACH, -1.0, 1.0))
GARBAGE_SCALE = float(np.clip(1.0 - GARBAGE_DECAY, 0.0, 1.0))
CAPACITY = INPUT_SIZE + N_UNITS
N_QUERIES = N_UNITS + OUTPUT_SIZE

NTILES = 8
TILE = CAPACITY // NTILES
C = CHANNEL_SIZE
E = ENV_DIMS


def _normalize(x):
    s = jnp.sum(x * x, axis=1, keepdims=True)
    return x * jax.lax.rsqrt(jnp.clip(s, 1e-24, None))


def _fused_kernel(state_ref, tails_ref, eh_ref, uh_ref, w_ref, b_ref,
                  db_ref, exit_ref, gsum_ref,
                  q_ref, dist_acc, gsum_acc):
    i = pl.program_id(0)

    @pl.when(i == 0)
    def _init_queries():
        q_ref[...] = jnp.concatenate(
            [_normalize(uh_ref[...]), _normalize(eh_ref[...])], axis=0)

    base = i * TILE
    rows = jax.lax.broadcasted_iota(jnp.int32, (TILE, 1), 0) + base
    valid = rows < INPUT_SIZE

    tp = jnp.where(valid, tails_ref[...], 0.0)
    zn = _normalize(tp)
    sp = jnp.where(valid, state_ref[...], 0.0)

    q = q_ref[...]
    sims = jax.lax.dot_general(
        q, zn, (((1,), (1,)), ((), ())),
        preferred_element_type=jnp.float32)
    infl = jnp.maximum(sims - REACH_THRESHOLD, 0.0)

    dist_part = jnp.dot(infl, sp, preferred_element_type=jnp.float32)

    t = sp * (jnp.sum(infl, axis=0) - 1.0)[:, None]
    db_ref[...] = sp - GARBAGE_SCALE * t
    g_part = -jnp.sum(t, axis=0, keepdims=True)

    @pl.when(i == 0)
    def _init():
        dist_acc[...] = dist_part
        gsum_acc[...] = g_part

    @pl.when(i > 0)
    def _accum():
        dist_acc[...] += dist_part
        gsum_acc[...] += g_part

    @pl.when(i == NTILES - 1)
    def _finish():
        dist = dist_acc[...]
        exit_ref[...] = dist[N_UNITS:]
        gsum_ref[...] = gsum_acc[...]
        unit_in = dist[:N_UNITS]
        w = w_ref[...]
        prod = jnp.sum(unit_in[:, :, None] * w, axis=1)
        unit_out = unit_in + jnp.maximum(prod + b_ref[...], 0.0)
        db_ref[TILE - N_UNITS:, :] = unit_out


@jax.jit
def _run(state, input_tails, exit_heads, unit_heads, unit_W, unit_b):
    out_shapes = (
        jax.ShapeDtypeStruct((CAPACITY, C), jnp.float32),
        jax.ShapeDtypeStruct((OUTPUT_SIZE, C), jnp.float32),
        jax.ShapeDtypeStruct((1, C), jnp.float32),
    )
    db_data, exit_out, gsum = pl.pallas_call(
        _fused_kernel,
        grid=(NTILES,),
        in_specs=[
            pl.BlockSpec((TILE, C), lambda i: (i, 0)),
            pl.BlockSpec((TILE, E), lambda i: (i, 0)),
            pl.BlockSpec((OUTPUT_SIZE, E), lambda i: (0, 0)),
            pl.BlockSpec((N_UNITS, E), lambda i: (0, 0)),
            pl.BlockSpec((N_UNITS, C, C), lambda i: (0, 0, 0)),
            pl.BlockSpec((N_UNITS, C), lambda i: (0, 0)),
        ],
        out_specs=[
            pl.BlockSpec((TILE, C), lambda i: (i, 0)),
            pl.BlockSpec((OUTPUT_SIZE, C), lambda i: (0, 0)),
            pl.BlockSpec((1, C), lambda i: (0, 0)),
        ],
        out_shape=out_shapes,
        scratch_shapes=[
            pltpu.VMEM((N_QUERIES, E), jnp.float32),
            pltpu.VMEM((N_QUERIES, C), jnp.float32),
            pltpu.VMEM((1, C), jnp.float32),
        ],
    )(state, input_tails, exit_heads, unit_heads, unit_W, unit_b)
    return exit_out, gsum.reshape(C), db_data


def kernel(state, input_tails, exit_heads, unit_heads, unit_tails, unit_W,
           unit_b, step=1):
    del unit_tails, step
    return _run(state, input_tails, exit_heads, unit_heads, unit_W, unit_b)


# speedup vs baseline: 1.4048x; 1.0180x over previous
import jax
import jax.numpy as jnp
import numpy as np
from jax.experimental import pallas as pl
from jax.experimental.pallas import tpu as pltpu

INPUT_SIZE = 16384
OUTPUT_SIZE = 1024
N_UNITS = 64
CHANNEL_SIZE = 64
ENV_DIMS = 32
REACH = 1.0
GARBAGE_DECAY = 0.9
REACH_THRESHOLD = float(np.clip(1.0 - REACH, -1.0, 1.0))
GARBAGE_SCALE = float(np.clip(1.0 - GARBAGE_DECAY, 0.0, 1.0))
CAPACITY = INPUT_SIZE + N_UNITS
N_QUERIES = N_UNITS + OUTPUT_SIZE

NTILES = 4
TILE = CAPACITY // NTILES
C = CHANNEL_SIZE
E = ENV_DIMS


def _normalize(x):
    s = jnp.sum(x * x, axis=1, keepdims=True)
    return x * jax.lax.rsqrt(jnp.clip(s, 1e-24, None))


def _fused_kernel(state_ref, tails_ref, eh_ref, uh_ref, w_ref, b_ref,
                  db_ref, exit_ref, gsum_ref,
                  q_ref, dist_acc, gsum_acc):
    i = pl.program_id(0)

    @pl.when(i == 0)
    def _init_queries():
        q_ref[...] = jnp.concatenate(
            [_normalize(uh_ref[...]), _normalize(eh_ref[...])], axis=0)

    base = i * TILE
    rows = jax.lax.broadcasted_iota(jnp.int32, (TILE, 1), 0) + base
    valid = rows < INPUT_SIZE

    tp = jnp.where(valid, tails_ref[...], 0.0)
    zn = _normalize(tp)
    sp = jnp.where(valid, state_ref[...], 0.0)

    q = q_ref[...]
    sims = jax.lax.dot_general(
        q, zn, (((1,), (1,)), ((), ())),
        preferred_element_type=jnp.float32)
    infl = jnp.maximum(sims - REACH_THRESHOLD, 0.0)

    dist_part = jnp.dot(infl, sp, preferred_element_type=jnp.float32)

    t = sp * (jnp.sum(infl, axis=0) - 1.0)[:, None]
    db_ref[...] = sp - GARBAGE_SCALE * t
    g_part = -jnp.sum(t, axis=0, keepdims=True)

    @pl.when(i == 0)
    def _init():
        dist_acc[...] = dist_part
        gsum_acc[...] = g_part

    @pl.when(i > 0)
    def _accum():
        dist_acc[...] += dist_part
        gsum_acc[...] += g_part

    @pl.when(i == NTILES - 1)
    def _finish():
        dist = dist_acc[...]
        exit_ref[...] = dist[N_UNITS:]
        gsum_ref[...] = gsum_acc[...]
        unit_in = dist[:N_UNITS]
        w = w_ref[...]
        prod = jnp.sum(unit_in[:, :, None] * w, axis=1)
        unit_out = unit_in + jnp.maximum(prod + b_ref[...], 0.0)
        db_ref[TILE - N_UNITS:, :] = unit_out


@jax.jit
def _run(state, input_tails, exit_heads, unit_heads, unit_W, unit_b):
    out_shapes = (
        jax.ShapeDtypeStruct((CAPACITY, C), jnp.float32),
        jax.ShapeDtypeStruct((OUTPUT_SIZE, C), jnp.float32),
        jax.ShapeDtypeStruct((1, C), jnp.float32),
    )
    db_data, exit_out, gsum = pl.pallas_call(
        _fused_kernel,
        grid=(NTILES,),
        in_specs=[
            pl.BlockSpec((TILE, C), lambda i: (i, 0)),
            pl.BlockSpec((TILE, E), lambda i: (i, 0)),
            pl.BlockSpec((OUTPUT_SIZE, E), lambda i: (0, 0)),
            pl.BlockSpec((N_UNITS, E), lambda i: (0, 0)),
            pl.BlockSpec((N_UNITS, C, C), lambda i: (0, 0, 0)),
            pl.BlockSpec((N_UNITS, C), lambda i: (0, 0)),
        ],
        out_specs=[
            pl.BlockSpec((TILE, C), lambda i: (i, 0)),
            pl.BlockSpec((OUTPUT_SIZE, C), lambda i: (0, 0)),
            pl.BlockSpec((1, C), lambda i: (0, 0)),
        ],
        out_shape=out_shapes,
        scratch_shapes=[
            pltpu.VMEM((N_QUERIES, E), jnp.float32),
            pltpu.VMEM((N_QUERIES, C), jnp.float32),
            pltpu.VMEM((1, C), jnp.float32),
        ],
    )(state, input_tails, exit_heads, unit_heads, unit_W, unit_b)
    return exit_out, gsum.reshape(C), db_data


def kernel(state, input_tails, exit_heads, unit_heads, unit_tails, unit_W,
           unit_b, step=1):
    del unit_tails, step
    return _run(state, input_tails, exit_heads, unit_heads, unit_W, unit_b)
